# baseline (speedup 1.0000x reference)
import jax
import jax.numpy as jnp
from jax import lax
from jax.experimental import pallas as pl
from jax.experimental.pallas import tpu as pltpu
from jax.experimental.pallas import tpu_sc as plsc

B, L = 4096, 200
N = B * L
D0, D1, D2 = 64, 32, 32
DO = D0 + D1 + D2
W = 32
R0, R1, R2 = 4096, 2048, 512
NC, NS = 2, 16
NW = NC * NS
IDR = 128
G = 2
C = G * IDR
K = 4 * C
TOK_PER_W = N // NW
CHUNKS_PER_W = TOK_PER_W // C
LANES = 16


def _encoder_kernel(ids0, ids1, ids2, t0, t1, t2, out,
                    tc_s,
                    ids0_v, ids1_v, ids2_v,
                    idx_v, out_v,
                    sem_i, sem_g, sem_w):
    cid = lax.axis_index("c")
    sid = lax.axis_index("s")
    wid = sid * NC + cid

    @pl.when(sid == 0)
    def _stage():
        pltpu.sync_copy(t0, tc_s.at[pl.ds(0, R0)])
        pltpu.sync_copy(t1, tc_s.at[pl.ds(R0, R1)])
        pltpu.sync_copy(t2, tc_s.at[pl.ds(R0 + R1, R2)])

    plsc.subcore_barrier()

    row0 = wid * CHUNKS_PER_W * G

    def fire_idx_fetch(c, p):
        r = row0 + G * lax.rem(c, CHUNKS_PER_W)
        pltpu.async_copy(ids0.at[pl.ds(r, G)], ids0_v.at[p], sem_i.at[p])
        pltpu.async_copy(ids1.at[pl.ds(r, G)], ids1_v.at[p], sem_i.at[p])
        pltpu.async_copy(ids2.at[pl.ds(r, G)], ids2_v.at[p], sem_i.at[p])

    def wait_idx_fetch(p):
        src = ids0.at[pl.ds(0, G)]
        pltpu.make_async_copy(src, ids0_v.at[p], sem_i.at[p]).wait()
        pltpu.make_async_copy(src, ids1_v.at[p], sem_i.at[p]).wait()
        pltpu.make_async_copy(src, ids2_v.at[p], sem_i.at[p]).wait()

    def compute_idx(p):
        lane = lax.iota(jnp.int32, LANES)
        tgt = idx_v.at[p]
        for j in range(G):
            for k in range(IDR // LANES):
                t16 = pl.ds(k * LANES, LANES)
                pos = 4 * (j * IDR + k * LANES + lane)
                v0 = ids0_v[p, j, t16]
                v1 = ids1_v[p, j, t16]
                v2 = ids2_v[p, j, t16]
                plsc.store_scatter(tgt, [pos], 2 * v0)
                plsc.store_scatter(tgt, [pos + 1], 2 * v0 + 1)
                plsc.store_scatter(tgt, [pos + 2], v1 + R0)
                plsc.store_scatter(tgt, [pos + 3], v2 + (R0 + R1))

    def wait_gather(p):
        pltpu.make_async_copy(tc_s.at[idx_v.at[p]], out_v.at[p],
                              sem_g.at[p]).wait()

    def fire_write(c, p):
        base = (wid * CHUNKS_PER_W + c) * K
        pltpu.async_copy(out_v.at[p], out.at[pl.ds(base, K)], sem_w.at[p])

    def wait_write(p):
        pltpu.make_async_copy(out_v.at[p], out.at[pl.ds(0, K)],
                              sem_w.at[p]).wait()

    def chunk_step(c, p):
        q = 1 - p
        @pl.when(c >= 2)
        def _():
            wait_write(p)
        pltpu.async_copy(tc_s.at[idx_v.at[p]], out_v.at[p], sem_g.at[p])

        @pl.when(c >= 1)
        def _():
            wait_gather(q)
            fire_write(c - 1, q)

        wait_idx_fetch(q)
        compute_idx(q)
        fire_idx_fetch(c + 2, p)

    fire_idx_fetch(0, 0)
    wait_idx_fetch(0)
    compute_idx(0)
    fire_idx_fetch(1, 1)

    def body(i, carry):
        chunk_step(2 * i, 0)
        chunk_step(2 * i + 1, 1)
        return carry

    lax.fori_loop(0, CHUNKS_PER_W // 2, body, 0)

    wait_gather(1)
    fire_write(CHUNKS_PER_W - 1, 1)
    wait_idx_fetch(1)
    wait_write(0)
    wait_write(1)


def kernel(pos_ids_0, pos_ids_1, pos_ids_2, table_0, table_1, table_2):
    ids0 = pos_ids_0.reshape(N // IDR, IDR)
    ids1 = pos_ids_1.reshape(N // IDR, IDR)
    ids2 = pos_ids_2.reshape(N // IDR, IDR)
    t0 = table_0.reshape(R0, W)

    mesh = plsc.VectorSubcoreMesh(core_axis_name="c", subcore_axis_name="s")
    run = pl.kernel(
        _encoder_kernel,
        out_type=jax.ShapeDtypeStruct((4 * N, W), jnp.float32),
        mesh=mesh,
        compiler_params=pltpu.CompilerParams(use_tc_tiling_on_sc=False,
                                             needs_layout_passes=False),
        scratch_types=[
            pltpu.VMEM_SHARED((R0 + R1 + R2, W), jnp.float32),
            pltpu.VMEM((2, G, IDR), jnp.int32),
            pltpu.VMEM((2, G, IDR), jnp.int32),
            pltpu.VMEM((2, G, IDR), jnp.int32),
            pltpu.VMEM((2, K), jnp.int32),
            pltpu.VMEM((2, K, W), jnp.float32),
            pltpu.SemaphoreType.DMA((2,)),
            pltpu.SemaphoreType.DMA((2,)),
            pltpu.SemaphoreType.DMA((2,)),
        ],
    )
    out = run(ids0, ids1, ids2, t0, table_1, table_2)
    return out.reshape(B, L, DO)
